# baseline (speedup 1.0000x reference)
import functools
import math

import jax
import jax.numpy as jnp
import numpy as np
from jax.experimental import pallas as pl
from jax.experimental.pallas import tpu as pltpu

_N_HEADS = 16
_FACTOR = 5
_HIGH = jax.lax.Precision.HIGHEST
_HPP = 2


def _threefry2x32(ks0, ks1, x0, x1):
    def rotl(x, r):
        return ((x << np.uint32(r)) | (x >> np.uint32(32 - r))).astype(
            np.uint32)
    rots = [[13, 15, 26, 6], [17, 29, 16, 24]]
    ks = [np.uint32(ks0), np.uint32(ks1),
          np.uint32(np.uint32(ks0) ^ np.uint32(ks1) ^ np.uint32(0x1BD11BDA))]
    x0 = (x0 + ks[0]).astype(np.uint32)
    x1 = (x1 + ks[1]).astype(np.uint32)
    for g in range(5):
        for r in rots[g % 2]:
            x0 = (x0 + x1).astype(np.uint32)
            x1 = rotl(x1, r)
            x1 = (x1 ^ x0).astype(np.uint32)
        x0 = (x0 + ks[(g + 1) % 3]).astype(np.uint32)
        x1 = (x1 + ks[(g + 2) % 3] + np.uint32(g + 1)).astype(np.uint32)
    return x0, x1


def _np_random_bits(kd, n):
    i = np.arange(n, dtype=np.uint64)
    c1 = (i >> np.uint64(32)).astype(np.uint32)
    c2 = (i & np.uint64(0xFFFFFFFF)).astype(np.uint32)
    x0, x1 = _threefry2x32(kd[0], kd[1], c1, c2)
    return (x0 ^ x1).astype(np.uint32)


@functools.lru_cache(maxsize=None)
def _sample_constants(L_Q: int, L_K: int, sample_k: int):
    k1 = _threefry2x32(0, 42, np.uint32(0), np.uint32(0))
    k2 = _threefry2x32(0, 42, np.uint32(0), np.uint32(1))
    n = L_Q * sample_k
    higher = _np_random_bits(k1, n).astype(np.uint64)
    lower = _np_random_bits(k2, n).astype(np.uint64)
    span = np.uint64(L_K)
    multiplier = ((2 ** 16 % L_K) ** 2) % L_K
    offset = ((higher % span) * np.uint64(multiplier) + lower % span) % span
    index_sample = offset.astype(np.int64).reshape(L_Q, sample_k)
    cnt = np.zeros((L_Q, L_K), dtype=np.float32)
    np.add.at(cnt, (np.arange(L_Q)[:, None], index_sample), 1.0)
    bias_t = np.where(cnt > 0, 0.0, -np.inf).astype(np.float32).T.copy()
    return (jnp.asarray(cnt, dtype=jnp.bfloat16),
            jnp.asarray(bias_t, dtype=jnp.float32))


def _measure_kernel(q_ref, k_ref, cnt_ref, bias_ref, m_ref, *, E, L_K,
                    n_chunks):
    chunk = L_K // n_chunks
    for hh in range(_HPP):
        q = q_ref[0, :, hh * E:(hh + 1) * E]
        k = k_ref[0, :, hh * E:(hh + 1) * E]
        qb = q.astype(jnp.bfloat16)
        kb = k.astype(jnp.bfloat16)
        L_Q = q.shape[0]
        ksum = jax.lax.dot_general(
            cnt_ref[...], kb, (((1,), (0,)), ((), ())),
            preferred_element_type=jnp.float32)
        prod = q * ksum
        ones = jnp.ones((1, E), dtype=jnp.float32)
        m2 = jax.lax.dot_general(
            ones, prod, (((1,), (1,)), ((), ())),
            preferred_element_type=jnp.float32, precision=_HIGH)
        m1 = jnp.full((1, L_Q), -jnp.inf, dtype=jnp.float32)
        for c in range(n_chunks):
            kc = kb[c * chunk:(c + 1) * chunk, :]
            s_c = jax.lax.dot_general(
                kc, qb, (((1,), (1,)), ((), ())),
                preferred_element_type=jnp.float32)
            biased = s_c + bias_ref[c * chunk:(c + 1) * chunk, :]
            m1 = jnp.maximum(m1, jnp.max(biased, axis=0, keepdims=True))
        m_ref[hh, 0, :] = (m1 - m2 * (1.0 / L_K))[0]


def _attend_kernel(q_ref, k_ref, v_ref, m_ref, out_ref, idx_scr, *, E,
                   scale, u, u_pad, G):
    i = pl.program_id(0)
    L_Q = q_ref.shape[1]

    @pl.when(i == 0)
    def _topk():
        x = m_ref[:, 0, :]
        iota_l = jax.lax.broadcasted_iota(jnp.int32, (G, L_Q), 1)
        col_t = jax.lax.broadcasted_iota(jnp.int32, (G, u_pad), 1)
        idx_acc = jnp.full((G, u_pad), -1, dtype=jnp.int32)
        for t in range(u):
            rmax = jnp.max(x, axis=1, keepdims=True)
            cand = jnp.where(x == rmax, iota_l, L_Q)
            arg = jnp.min(cand, axis=1, keepdims=True)
            idx_acc = jnp.where(col_t == t, arg, idx_acc)
            x = jnp.where(iota_l == arg, -jnp.inf, x)
        idx_scr[...] = idx_acc

    for hh in range(_HPP):
        q = q_ref[0, :, hh * E:(hh + 1) * E]
        k = k_ref[0, :, hh * E:(hh + 1) * E]
        v = v_ref[0, :, hh * E:(hh + 1) * E]
        idx = idx_scr[pl.ds(_HPP * i + hh, 1), :]
        L_K = k.shape[0]
        oh = (jax.lax.broadcasted_iota(jnp.int32, (u_pad, L_Q), 1)
              == idx.reshape(u_pad, 1)).astype(jnp.bfloat16)
        q_sel = jax.lax.dot_general(
            oh, q.astype(jnp.bfloat16), (((1,), (0,)), ((), ())),
            preferred_element_type=jnp.float32)
        s = jax.lax.dot_general(
            q_sel.astype(jnp.bfloat16), k.astype(jnp.bfloat16),
            (((1,), (1,)), ((), ())),
            preferred_element_type=jnp.float32) * scale
        smax = jnp.max(s, axis=1, keepdims=True)
        e = jnp.exp(s - smax)
        p = e / jnp.sum(e, axis=1, keepdims=True)
        upd = jax.lax.dot_general(
            p.astype(jnp.bfloat16), v.astype(jnp.bfloat16),
            (((1,), (0,)), ((), ())),
            preferred_element_type=jnp.float32)
        oh_t = (jax.lax.broadcasted_iota(jnp.int32, (L_Q, u_pad), 0)
                == idx).astype(jnp.float32)
        p2 = jax.lax.dot_general(
            oh_t, upd, (((1,), (0,)), ((), ())),
            preferred_element_type=jnp.float32, precision=_HIGH)
        sel = jnp.sum(oh_t, axis=1, keepdims=True)
        vmean = jnp.mean(v, axis=0, keepdims=True)
        out_ref[0, :, hh * E:(hh + 1) * E] = vmean * (1.0 - sel) + p2


def kernel(queries, keys, values):
    B, L_Q, D = queries.shape
    L_K = keys.shape[1]
    H = _N_HEADS
    E = D // H
    U_part = _FACTOR * int(np.ceil(np.log(max(L_K, 1))))
    u = _FACTOR * int(np.ceil(np.log(max(L_Q, 1))))
    U_part = max(1, min(U_part, L_K))
    u = max(1, min(u, L_Q))
    sample_k = min(U_part, L_K)
    cnt, bias_t = _sample_constants(L_Q, L_K, sample_k)
    G = B * H
    GP = G // _HPP
    PPB = H // _HPP
    n_chunks = 8
    u_pad = ((u + 63) // 64) * 64

    qkv_spec = pl.BlockSpec((1, L_Q, _HPP * E),
                            lambda i: (i // PPB, 0, i % PPB))

    m = pl.pallas_call(
        functools.partial(_measure_kernel, E=E, L_K=L_K, n_chunks=n_chunks),
        grid=(GP,),
        in_specs=[
            qkv_spec,
            qkv_spec,
            pl.BlockSpec((L_Q, L_K), lambda i: (0, 0)),
            pl.BlockSpec((L_K, L_Q), lambda i: (0, 0)),
        ],
        out_specs=pl.BlockSpec((_HPP, 1, L_Q), lambda i: (i, 0, 0)),
        out_shape=jax.ShapeDtypeStruct((G, 1, L_Q), jnp.float32),
    )(queries, keys, cnt, bias_t)

    out = pl.pallas_call(
        functools.partial(_attend_kernel, E=E, scale=1.0 / math.sqrt(E),
                          u=u, u_pad=u_pad, G=G),
        grid=(GP,),
        in_specs=[
            qkv_spec,
            qkv_spec,
            qkv_spec,
            pl.BlockSpec((G, 1, L_Q), lambda i: (0, 0, 0)),
        ],
        out_specs=qkv_spec,
        out_shape=jax.ShapeDtypeStruct((B, L_Q, D), jnp.float32),
        scratch_shapes=[pltpu.VMEM((G, u_pad), jnp.int32)],
    )(queries, keys, values, m)
    return out.reshape(B, L_Q, H, E)


# speedup vs baseline: 1.7165x; 1.7165x over previous
import functools
import math

import jax
import jax.numpy as jnp
import numpy as np
from jax.experimental import pallas as pl
from jax.experimental.pallas import tpu as pltpu

_N_HEADS = 16
_FACTOR = 5
_HIGH = jax.lax.Precision.HIGHEST
_HPP = 2


def _threefry2x32(ks0, ks1, x0, x1):
    def rotl(x, r):
        return ((x << np.uint32(r)) | (x >> np.uint32(32 - r))).astype(
            np.uint32)
    rots = [[13, 15, 26, 6], [17, 29, 16, 24]]
    ks = [np.uint32(ks0), np.uint32(ks1),
          np.uint32(np.uint32(ks0) ^ np.uint32(ks1) ^ np.uint32(0x1BD11BDA))]
    x0 = (x0 + ks[0]).astype(np.uint32)
    x1 = (x1 + ks[1]).astype(np.uint32)
    for g in range(5):
        for r in rots[g % 2]:
            x0 = (x0 + x1).astype(np.uint32)
            x1 = rotl(x1, r)
            x1 = (x1 ^ x0).astype(np.uint32)
        x0 = (x0 + ks[(g + 1) % 3]).astype(np.uint32)
        x1 = (x1 + ks[(g + 2) % 3] + np.uint32(g + 1)).astype(np.uint32)
    return x0, x1


def _np_random_bits(kd, n):
    i = np.arange(n, dtype=np.uint64)
    c1 = (i >> np.uint64(32)).astype(np.uint32)
    c2 = (i & np.uint64(0xFFFFFFFF)).astype(np.uint32)
    x0, x1 = _threefry2x32(kd[0], kd[1], c1, c2)
    return (x0 ^ x1).astype(np.uint32)


@functools.lru_cache(maxsize=None)
def _sample_constants(L_Q: int, L_K: int, sample_k: int):
    k1 = _threefry2x32(0, 42, np.uint32(0), np.uint32(0))
    k2 = _threefry2x32(0, 42, np.uint32(0), np.uint32(1))
    n = L_Q * sample_k
    higher = _np_random_bits(k1, n).astype(np.uint64)
    lower = _np_random_bits(k2, n).astype(np.uint64)
    span = np.uint64(L_K)
    multiplier = ((2 ** 16 % L_K) ** 2) % L_K
    offset = ((higher % span) * np.uint64(multiplier) + lower % span) % span
    index_sample = offset.astype(np.int64).reshape(L_Q, sample_k)
    cnt = np.zeros((L_Q, L_K), dtype=np.float32)
    np.add.at(cnt, (np.arange(L_Q)[:, None], index_sample), 1.0)
    bias = np.where(cnt > 0, 0.0, -np.inf).astype(np.float32)
    return (jnp.asarray(cnt.T.copy(), dtype=jnp.float32),
            jnp.asarray(bias.T.copy(), dtype=jnp.float32))


def _measure_kernel(q_ref, k_ref, cnt_ref, bias_ref, m_ref, *, E, L_K,
                    n_chunks):
    chunk = L_K // n_chunks
    for hh in range(_HPP):
        q = q_ref[0, :, hh * E:(hh + 1) * E]
        k = k_ref[0, :, hh * E:(hh + 1) * E]
        qb = q.astype(jnp.bfloat16)
        kb = k.astype(jnp.bfloat16)
        L_Q = q.shape[0]
        m1 = jnp.full((1, L_Q), -jnp.inf, dtype=jnp.float32)
        msum = jnp.zeros((1, L_Q), dtype=jnp.float32)
        for c in range(n_chunks):
            kc = kb[c * chunk:(c + 1) * chunk, :]
            s_c = jax.lax.dot_general(
                kc, qb, (((1,), (1,)), ((), ())),
                preferred_element_type=jnp.float32)
            biased = s_c + bias_ref[c * chunk:(c + 1) * chunk, :]
            m1 = jnp.maximum(m1, jnp.max(biased, axis=0, keepdims=True))
            msum = msum + jnp.sum(
                s_c * cnt_ref[c * chunk:(c + 1) * chunk, :], axis=0,
                keepdims=True)
        m_ref[hh, 0, :] = (m1 - msum * (1.0 / L_K))[0]


def _topk_kernel(m_ref, idx_ref, *, u, u_pad):
    G, _, L_Q = m_ref.shape
    x = m_ref[:, 0, :]
    iota_l = jax.lax.broadcasted_iota(jnp.int32, (G, L_Q), 1)
    col_t = jax.lax.broadcasted_iota(jnp.int32, (G, u_pad), 1)
    idx_acc = jnp.full((G, u_pad), -1, dtype=jnp.int32)
    for t in range(u):
        rmax = jnp.max(x, axis=1, keepdims=True)
        cand = jnp.where(x == rmax, iota_l, L_Q)
        arg = jnp.min(cand, axis=1, keepdims=True)
        idx_acc = jnp.where(col_t == t, arg, idx_acc)
        x = jnp.where(iota_l == arg, -jnp.inf, x)
    idx_ref[...] = idx_acc


def _attend_kernel(q_ref, k_ref, v_ref, idx_all_ref, out_ref, *, E,
                   scale, u_pad):
    i = pl.program_id(0)
    for hh in range(_HPP):
        q = q_ref[0, :, hh * E:(hh + 1) * E]
        k = k_ref[0, :, hh * E:(hh + 1) * E]
        v = v_ref[0, :, hh * E:(hh + 1) * E]
        L_Q = q.shape[0]
        idx = idx_all_ref[pl.ds(_HPP * i + hh, 1), :]
        L_K = k.shape[0]
        oh = (jax.lax.broadcasted_iota(jnp.int32, (u_pad, L_Q), 1)
              == idx.reshape(u_pad, 1)).astype(jnp.bfloat16)
        q_sel = jax.lax.dot_general(
            oh, q.astype(jnp.bfloat16), (((1,), (0,)), ((), ())),
            preferred_element_type=jnp.float32)
        s = jax.lax.dot_general(
            q_sel.astype(jnp.bfloat16), k.astype(jnp.bfloat16),
            (((1,), (1,)), ((), ())),
            preferred_element_type=jnp.float32) * scale
        smax = jnp.max(s, axis=1, keepdims=True)
        e = jnp.exp(s - smax)
        p = e / jnp.sum(e, axis=1, keepdims=True)
        upd = jax.lax.dot_general(
            p.astype(jnp.bfloat16), v.astype(jnp.bfloat16),
            (((1,), (0,)), ((), ())),
            preferred_element_type=jnp.float32)
        oh_t = (jax.lax.broadcasted_iota(jnp.int32, (L_Q, u_pad), 0)
                == idx).astype(jnp.float32)
        p2 = jax.lax.dot_general(
            oh_t, upd, (((1,), (0,)), ((), ())),
            preferred_element_type=jnp.float32, precision=_HIGH)
        sel = jnp.sum(oh_t, axis=1, keepdims=True)
        vmean = jnp.mean(v, axis=0, keepdims=True)
        out_ref[0, :, hh * E:(hh + 1) * E] = vmean * (1.0 - sel) + p2


def kernel(queries, keys, values):
    B, L_Q, D = queries.shape
    L_K = keys.shape[1]
    H = _N_HEADS
    E = D // H
    U_part = _FACTOR * int(np.ceil(np.log(max(L_K, 1))))
    u = _FACTOR * int(np.ceil(np.log(max(L_Q, 1))))
    U_part = max(1, min(U_part, L_K))
    u = max(1, min(u, L_Q))
    sample_k = min(U_part, L_K)
    cnt, bias_t = _sample_constants(L_Q, L_K, sample_k)
    G = B * H
    GP = G // _HPP
    PPB = H // _HPP
    n_chunks = 8
    u_pad = ((u + 63) // 64) * 64

    qkv_spec = pl.BlockSpec((1, L_Q, _HPP * E),
                            lambda i: (i // PPB, 0, i % PPB))

    m = pl.pallas_call(
        functools.partial(_measure_kernel, E=E, L_K=L_K, n_chunks=n_chunks),
        grid=(GP,),
        in_specs=[
            qkv_spec,
            qkv_spec,
            pl.BlockSpec((L_Q, L_K), lambda i: (0, 0)),
            pl.BlockSpec((L_K, L_Q), lambda i: (0, 0)),
        ],
        out_specs=pl.BlockSpec((_HPP, 1, L_Q), lambda i: (i, 0, 0)),
        out_shape=jax.ShapeDtypeStruct((G, 1, L_Q), jnp.float32),
    )(queries, keys, cnt, bias_t)

    idx = pl.pallas_call(
        functools.partial(_topk_kernel, u=u, u_pad=u_pad),
        in_specs=[pl.BlockSpec((G, 1, L_Q), lambda: (0, 0, 0))],
        out_specs=pl.BlockSpec((G, u_pad), lambda: (0, 0)),
        grid=(),
        out_shape=jax.ShapeDtypeStruct((G, u_pad), jnp.int32),
    )(m)

    out = pl.pallas_call(
        functools.partial(_attend_kernel, E=E, scale=1.0 / math.sqrt(E),
                          u_pad=u_pad),
        grid=(GP,),
        in_specs=[
            qkv_spec,
            qkv_spec,
            qkv_spec,
            pl.BlockSpec((G, u_pad), lambda i: (0, 0)),
        ],
        out_specs=qkv_spec,
        out_shape=jax.ShapeDtypeStruct((B, L_Q, D), jnp.float32),
    )(queries, keys, values, idx)
    return out.reshape(B, L_Q, H, E)
